# baseline (speedup 1.0000x reference)
import functools

import jax
import jax.numpy as jnp
from jax import lax
from jax.experimental import pallas as pl
from jax.experimental.pallas import tpu as pltpu
from jax.experimental.pallas import tpu_sc as plsc

_M = 4
_V = 100000
_D = 32
_B = 4096
_T = 20
_ROWS = _M * _B * _T
_NW = 32
_RPW = _ROWS // _NW
_CHUNK = 256
_CPR = 5
_RROWS = _CPR * _CHUNK
_NR = _RPW // _RROWS
_NBUF = 2
_LANES = 16


def _gather_body(idx_hbm, w_hbm, out_hbm, idx_v, buf0, buf1,
                 gsem0, gsem1, osem0, osem1):
    c = lax.axis_index("c")
    s = lax.axis_index("s")
    wid = s * 2 + c
    base = wid * _RPW
    pltpu.sync_copy(idx_hbm.at[pl.ds(base, _RPW)], idx_v)

    off = (base // (_B * _T)) * _V

    def add_off(i, carry):
        for u in range(4):
            sl = pl.ds((i * 4 + u) * _LANES, _LANES)
            idx_v[sl] = idx_v[sl] + off
        return carry

    lax.fori_loop(0, _RPW // (4 * _LANES), add_off, 0)

    def fire_gathers(r, buf, gsem):
        for j in range(_CPR):
            k = r * _RROWS + j * _CHUNK
            pltpu.async_copy(
                w_hbm.at[idx_v.at[pl.ds(k, _CHUNK)]],
                buf.at[pl.ds(j * _CHUNK, _CHUNK)], gsem)

    def drain(buf, sem):
        pltpu.make_async_copy(
            out_hbm.at[pl.ds(0, _RROWS)], buf, sem).wait()

    fire_gathers(0, buf0, gsem0)
    fire_gathers(1, buf1, gsem1)

    def body(i, carry):
        for half, buf, gsem, osem in (
                (0, buf0, gsem0, osem0), (1, buf1, gsem1, osem1)):
            r = i * _NBUF + half
            drain(buf, gsem)
            pltpu.async_copy(
                buf, out_hbm.at[pl.ds(base + r * _RROWS, _RROWS)], osem)
            drain(buf, osem)

            @pl.when(r < _NR - _NBUF)
            def _():
                fire_gathers(r + _NBUF, buf, gsem)
        return carry

    lax.fori_loop(0, _NR // _NBUF, body, 0)


@jax.jit
def _run(idx_flat, w_flat):
    mesh = plsc.VectorSubcoreMesh(core_axis_name="c", subcore_axis_name="s")
    f = functools.partial(
        pl.kernel,
        mesh=mesh,
        out_type=jax.ShapeDtypeStruct((_ROWS, _D), jnp.float32),
        scratch_types=[
            pltpu.VMEM((_RPW,), jnp.int32),
            pltpu.VMEM((_RROWS, _D), jnp.float32),
            pltpu.VMEM((_RROWS, _D), jnp.float32),
            pltpu.SemaphoreType.DMA,
            pltpu.SemaphoreType.DMA,
            pltpu.SemaphoreType.DMA,
            pltpu.SemaphoreType.DMA,
        ],
        compiler_params=pltpu.CompilerParams(use_tc_tiling_on_sc=False),
    )(_gather_body)
    return f(idx_flat, w_flat)


def kernel(idx, weight):
    idx_flat = idx.reshape(_ROWS).astype(jnp.int32)
    w_v = jnp.swapaxes(weight, 1, 2)
    w_rm = jnp.transpose(w_v, (0, 2, 1))
    w_flat = w_rm.reshape(_M * _V, _D)
    out = _run(idx_flat, w_flat)
    o = out.reshape(_M, _B, _T, _D)
    o_t = jnp.transpose(o, (0, 2, 3, 1))
    return jnp.transpose(o_t, (0, 3, 1, 2))
